# baseline (speedup 1.0000x reference)
import functools

import jax
import jax.numpy as jnp
from jax import lax
from jax.experimental import pallas as pl
from jax.experimental.pallas import tpu as pltpu
from jax.experimental.pallas import tpu_sc as plsc

N_NODES = 10000
N_EDGES = 320000
D = 128

NC = 2
NS = 16
NW = NC * NS
CHUNK = 96
N_CHUNKS = 105
E_PER_W = N_CHUNKS * CHUNK
E_PAD = NW * E_PER_W
N_PAD = 10240
ROWS_PER_TILE = N_PAD // NS

_mesh = plsc.VectorSubcoreMesh(core_axis_name="c", subcore_axis_name="s")

_SC_OUT_TYPE = [
    jax.ShapeDtypeStruct((NC * N_PAD, D), jnp.float32),
    jax.ShapeDtypeStruct((NW * N_PAD,), jnp.float32),
]
_SC_SCRATCH = [
    pltpu.VMEM((CHUNK,), jnp.int32),
    pltpu.VMEM((CHUNK,), jnp.int32),
    pltpu.VMEM((CHUNK,), jnp.int32),
    pltpu.VMEM((CHUNK,), jnp.int32),
    pltpu.VMEM((CHUNK, D), jnp.float32),
    pltpu.VMEM((CHUNK, D), jnp.float32),
    pltpu.VMEM((N_PAD,), jnp.float32),
    pltpu.VMEM_SHARED((N_PAD, D), jnp.float32),
    pltpu.SemaphoreType.DMA,
    pltpu.SemaphoreType.DMA,
    pltpu.SemaphoreType.DMA,
    pltpu.SemaphoreType.DMA,
]


def _sc_aggregate_body(src_hbm, dst_hbm, x_hbm, acc_out, deg_out,
                       src_v0, dst_v0, src_v1, dst_v1, rows_v0, rows_v1,
                       deg_loc, acc_sh, semg0, semg1, semi0, semi1):
    cid = lax.axis_index("c")
    sid = lax.axis_index("s")
    wid = sid * NC + cid

    zeros16 = jnp.zeros((16,), jnp.float32)

    src_v = (src_v0, src_v1)
    dst_v = (dst_v0, dst_v1)
    rows_v = (rows_v0, rows_v1)
    semg = (semg0, semg1)
    semi = (semi0, semi1)

    def _fill_z(r, _):
        for c in range(D // 16):
            rows_v0[r, pl.ds(c * 16, 16)] = zeros16
        return 0
    lax.fori_loop(0, CHUNK, _fill_z, 0)

    def _fill_zdeg(r, _):
        deg_loc[pl.ds(r * 16, 16)] = zeros16
        return 0
    lax.fori_loop(0, N_PAD // 16, _fill_zdeg, 0)

    r0 = sid * ROWS_PER_TILE
    ZB = 64
    for j in range(ROWS_PER_TILE // ZB):
        pltpu.sync_copy(rows_v0.at[pl.ds(0, ZB)],
                        acc_sh.at[pl.ds(r0 + j * ZB, ZB)])
    plsc.subcore_barrier()

    base = wid * E_PER_W

    def _start_idx(j, b):
        off = base + j * CHUNK
        pltpu.async_copy(src_hbm.at[pl.ds(off, CHUNK)], src_v[b], semi[b])
        pltpu.async_copy(dst_hbm.at[pl.ds(off, CHUNK)], dst_v[b], semi[b])

    def _wait_idx(b):
        pltpu.make_async_copy(src_hbm.at[pl.ds(0, CHUNK)], src_v[b],
                              semi[b]).wait()
        pltpu.make_async_copy(dst_hbm.at[pl.ds(0, CHUNK)], dst_v[b],
                              semi[b]).wait()

    def _start_gather(b):
        pltpu.async_copy(x_hbm.at[src_v[b]], rows_v[b], semg[b])

    def _wait_gather(b):
        pltpu.make_async_copy(x_hbm.at[src_v[b]], rows_v[b], semg[b]).wait()

    def _finish_chunk(b):
        pltpu.sync_copy(rows_v[b], acc_sh.at[dst_v[b]], add=True)
        for i in range(CHUNK // 16):
            idx = dst_v[b][pl.ds(i * 16, 16)]
            cnt, last = plsc.scan_count(idx)
            plsc.addupdate_scatter(deg_loc, [idx], cnt.astype(jnp.float32),
                                   mask=last)

    _start_idx(0, 0)
    _wait_idx(0)
    _start_gather(0)
    _start_idx(1, 1)

    def _step(j, b, prefetch):
        _wait_gather(b)
        _wait_idx(1 - b)
        _start_gather(1 - b)
        _finish_chunk(b)

        @pl.when(prefetch)
        def _():
            _start_idx(j + 2, b)

    def _pair(t, _):
        j = 2 * t
        _step(j, 0, jnp.bool_(True))
        _step(j + 1, 1, t < (N_CHUNKS - 1) // 2 - 1)
        return 0

    lax.fori_loop(0, (N_CHUNKS - 1) // 2, _pair, 0)

    _wait_gather(0)
    _finish_chunk(0)
    plsc.subcore_barrier()

    out0 = cid * N_PAD + r0
    for j in range(ROWS_PER_TILE // ZB):
        pltpu.sync_copy(acc_sh.at[pl.ds(r0 + j * ZB, ZB)],
                        rows_v0.at[pl.ds(0, ZB)])
        pltpu.sync_copy(rows_v0.at[pl.ds(0, ZB)],
                        acc_out.at[pl.ds(out0 + j * ZB, ZB)])
    pltpu.sync_copy(deg_loc, deg_out.at[pl.ds(wid * N_PAD, N_PAD)])


_sc_aggregate = pl.kernel(
    _sc_aggregate_body,
    out_type=_SC_OUT_TYPE,
    mesh=_mesh,
    scratch_types=_SC_SCRATCH,
    compiler_params=pltpu.CompilerParams(needs_layout_passes=False),
)


TC_BLK = 1024


def _tc_finish_body(acc_ref, deg_ref, w_ref, b_ref, out_ref):
    a = acc_ref[0] + acc_ref[1]
    d = jnp.sum(deg_ref[...], axis=0)[:, None]
    mean = a / jnp.maximum(d, 1.0)
    out_ref[...] = (
        jnp.dot(mean, w_ref[...], preferred_element_type=jnp.float32)
        + b_ref[...]
    )


def _tc_finish(acc, deg, weight, bias2d):
    grid = (N_PAD // TC_BLK,)
    return pl.pallas_call(
        _tc_finish_body,
        grid=grid,
        in_specs=[
            pl.BlockSpec((NC, TC_BLK, D), lambda i: (0, i, 0)),
            pl.BlockSpec((NW, TC_BLK), lambda i: (0, i)),
            pl.BlockSpec((D, D), lambda i: (0, 0)),
            pl.BlockSpec((1, D), lambda i: (0, 0)),
        ],
        out_specs=pl.BlockSpec((TC_BLK, D), lambda i: (i, 0)),
        out_shape=jax.ShapeDtypeStruct((N_PAD, D), jnp.float32),
    )(acc, deg, weight, bias2d)


@jax.jit
def kernel(x, edge_index, weight, bias):
    npad = E_PAD - N_EDGES
    dst = jnp.concatenate(
        [edge_index[0], jnp.full((npad,), N_NODES, jnp.int32)])
    src = jnp.concatenate([edge_index[1], jnp.zeros((npad,), jnp.int32)])
    acc, deg = _sc_aggregate(src, dst, x)
    acc = acc.reshape(NC, N_PAD, D)
    deg = deg.reshape(NW, N_PAD)
    out = _tc_finish(acc, deg, weight, bias.reshape(1, D))
    return out[:N_NODES]


# speedup vs baseline: 1.5313x; 1.5313x over previous
import functools

import jax
import jax.numpy as jnp
from jax import lax
from jax.experimental import pallas as pl
from jax.experimental.pallas import tpu as pltpu
from jax.experimental.pallas import tpu_sc as plsc

N_NODES = 10000
N_EDGES = 320000
D = 128

NC = 2
NS = 16
NW = NC * NS
CHUNK = 80
N_CHUNKS = 125
E_PER_W = N_CHUNKS * CHUNK
E_PAD = NW * E_PER_W
N_PAD = 10240
ROWS_PER_TILE = N_PAD // NS

_mesh = plsc.VectorSubcoreMesh(core_axis_name="c", subcore_axis_name="s")

_SC_OUT_TYPE = [
    jax.ShapeDtypeStruct((NC * N_PAD, D), jnp.float32),
    jax.ShapeDtypeStruct((NW * N_PAD,), jnp.float32),
]
_SC_SCRATCH = [
    pltpu.VMEM((CHUNK,), jnp.int32),
    pltpu.VMEM((CHUNK,), jnp.int32),
    pltpu.VMEM((CHUNK,), jnp.int32),
    pltpu.VMEM((CHUNK,), jnp.int32),
    pltpu.VMEM((CHUNK, D), jnp.float32),
    pltpu.VMEM((CHUNK, D), jnp.float32),
    pltpu.VMEM((N_PAD,), jnp.float32),
    pltpu.VMEM_SHARED((N_PAD, D), jnp.float32),
    pltpu.SemaphoreType.DMA,
    pltpu.SemaphoreType.DMA,
    pltpu.SemaphoreType.DMA,
    pltpu.SemaphoreType.DMA,
]


def _sc_aggregate_body(src_hbm, dst_hbm, x_hbm, acc_out, deg_out,
                       src_v0, dst_v0, src_v1, dst_v1, rows_v0, rows_v1,
                       deg_loc, acc_sh, semg0, semg1, semi0, semi1):
    cid = lax.axis_index("c")
    sid = lax.axis_index("s")
    wid = sid * NC + cid

    zeros16 = jnp.zeros((16,), jnp.float32)

    src_v = (src_v0, src_v1)
    dst_v = (dst_v0, dst_v1)
    rows_v = (rows_v0, rows_v1)
    semg = (semg0, semg1)
    semi = (semi0, semi1)

    def _fill_z(r, _):
        for c in range(D // 16):
            rows_v0[r, pl.ds(c * 16, 16)] = zeros16
        return 0
    lax.fori_loop(0, CHUNK, _fill_z, 0)

    def _fill_zdeg(r, _):
        deg_loc[pl.ds(r * 16, 16)] = zeros16
        return 0
    lax.fori_loop(0, N_PAD // 16, _fill_zdeg, 0)

    r0 = sid * ROWS_PER_TILE
    ZB = 64
    for j in range(ROWS_PER_TILE // ZB):
        pltpu.sync_copy(rows_v0.at[pl.ds(0, ZB)],
                        acc_sh.at[pl.ds(r0 + j * ZB, ZB)])
    plsc.subcore_barrier()

    base = wid * E_PER_W

    def _start_idx(j, b):
        off = base + j * CHUNK
        pltpu.async_copy(src_hbm.at[pl.ds(off, CHUNK)], src_v[b], semi[b])
        pltpu.async_copy(dst_hbm.at[pl.ds(off, CHUNK)], dst_v[b], semi[b])

    def _wait_idx(b):
        pltpu.make_async_copy(src_hbm.at[pl.ds(0, CHUNK)], src_v[b],
                              semi[b]).wait()
        pltpu.make_async_copy(dst_hbm.at[pl.ds(0, CHUNK)], dst_v[b],
                              semi[b]).wait()

    def _start_gather(b):
        pltpu.async_copy(x_hbm.at[src_v[b]], rows_v[b], semg[b])

    def _wait_gather(b):
        pltpu.make_async_copy(x_hbm.at[src_v[b]], rows_v[b], semg[b]).wait()

    def _finish_chunk(b):
        pltpu.sync_copy(rows_v[b], acc_sh.at[dst_v[b]], add=True)
        for i in range(CHUNK // 16):
            idx = dst_v[b][pl.ds(i * 16, 16)]
            cnt, last = plsc.scan_count(idx)
            plsc.addupdate_scatter(deg_loc, [idx], cnt.astype(jnp.float32),
                                   mask=last)

    _start_idx(0, 0)
    _wait_idx(0)
    _start_gather(0)
    _start_idx(1, 1)

    def _step(j, b, prefetch):
        _wait_gather(b)
        _wait_idx(1 - b)
        _start_gather(1 - b)
        _finish_chunk(b)

        @pl.when(prefetch)
        def _():
            _start_idx(j + 2, b)

    def _pair(t, _):
        j = 2 * t
        _step(j, 0, jnp.bool_(True))
        _step(j + 1, 1, t < (N_CHUNKS - 1) // 2 - 1)
        return 0

    lax.fori_loop(0, (N_CHUNKS - 1) // 2, _pair, 0)

    _wait_gather(0)
    _finish_chunk(0)
    plsc.subcore_barrier()

    out0 = cid * N_PAD + r0
    for j in range(ROWS_PER_TILE // ZB):
        pltpu.sync_copy(acc_sh.at[pl.ds(r0 + j * ZB, ZB)],
                        rows_v0.at[pl.ds(0, ZB)])
        pltpu.sync_copy(rows_v0.at[pl.ds(0, ZB)],
                        acc_out.at[pl.ds(out0 + j * ZB, ZB)])
    pltpu.sync_copy(deg_loc, deg_out.at[pl.ds(wid * N_PAD, N_PAD)])


_sc_aggregate = pl.kernel(
    _sc_aggregate_body,
    out_type=_SC_OUT_TYPE,
    mesh=_mesh,
    scratch_types=_SC_SCRATCH,
    compiler_params=pltpu.CompilerParams(needs_layout_passes=False),
)


TC_BLK = 1024


def _tc_finish_body(acc_ref, deg_ref, w_ref, b_ref, out_ref):
    a = acc_ref[0] + acc_ref[1]
    d = jnp.sum(deg_ref[...], axis=0)[:, None]
    mean = a / jnp.maximum(d, 1.0)
    out_ref[...] = (
        jnp.dot(mean, w_ref[...], preferred_element_type=jnp.float32)
        + b_ref[...]
    )


def _tc_finish(acc, deg, weight, bias2d):
    grid = (N_PAD // TC_BLK,)
    return pl.pallas_call(
        _tc_finish_body,
        grid=grid,
        in_specs=[
            pl.BlockSpec((NC, TC_BLK, D), lambda i: (0, i, 0)),
            pl.BlockSpec((NW, TC_BLK), lambda i: (0, i)),
            pl.BlockSpec((D, D), lambda i: (0, 0)),
            pl.BlockSpec((1, D), lambda i: (0, 0)),
        ],
        out_specs=pl.BlockSpec((TC_BLK, D), lambda i: (i, 0)),
        out_shape=jax.ShapeDtypeStruct((N_PAD, D), jnp.float32),
    )(acc, deg, weight, bias2d)


@jax.jit
def kernel(x, edge_index, weight, bias):
    npad = E_PAD - N_EDGES
    if npad:
        dst = jnp.concatenate(
            [edge_index[0], jnp.full((npad,), N_NODES, jnp.int32)])
        src = jnp.concatenate([edge_index[1], jnp.zeros((npad,), jnp.int32)])
    else:
        dst = edge_index[0]
        src = edge_index[1]
    acc, deg = _sc_aggregate(src, dst, x)
    acc = acc.reshape(NC, N_PAD, D)
    deg = deg.reshape(NW, N_PAD)
    out = _tc_finish(acc, deg, weight, bias.reshape(1, D))
    return out[:N_NODES]


# speedup vs baseline: 1.5377x; 1.0042x over previous
import functools

import jax
import jax.numpy as jnp
from jax import lax
from jax.experimental import pallas as pl
from jax.experimental.pallas import tpu as pltpu
from jax.experimental.pallas import tpu_sc as plsc

N_NODES = 10000
N_EDGES = 320000
D = 128

NC = 2
NS = 16
NW = NC * NS
CHUNK = 80
N_CHUNKS = 125
E_PER_W = N_CHUNKS * CHUNK
E_PAD = NW * E_PER_W
N_PAD = 10240
ROWS_PER_TILE = N_PAD // NS

_mesh = plsc.VectorSubcoreMesh(core_axis_name="c", subcore_axis_name="s")

_SC_OUT_TYPE = [
    jax.ShapeDtypeStruct((NC * N_PAD, D), jnp.float32),
    jax.ShapeDtypeStruct((NW * N_PAD,), jnp.float32),
]
_SC_SCRATCH = [
    pltpu.VMEM((CHUNK,), jnp.int32),
    pltpu.VMEM((CHUNK,), jnp.int32),
    pltpu.VMEM((CHUNK,), jnp.int32),
    pltpu.VMEM((CHUNK,), jnp.int32),
    pltpu.VMEM((CHUNK, D), jnp.float32),
    pltpu.VMEM((CHUNK, D), jnp.float32),
    pltpu.VMEM((N_PAD,), jnp.float32),
    pltpu.VMEM_SHARED((N_PAD, D), jnp.float32),
    pltpu.SemaphoreType.DMA,
    pltpu.SemaphoreType.DMA,
    pltpu.SemaphoreType.DMA,
    pltpu.SemaphoreType.DMA,
    pltpu.SemaphoreType.DMA,
    pltpu.SemaphoreType.DMA,
]


def _sc_aggregate_body(src_hbm, dst_hbm, x_hbm, acc_out, deg_out,
                       src_v0, dst_v0, src_v1, dst_v1, rows_v0, rows_v1,
                       deg_loc, acc_sh, semg0, semg1, semi0, semi1,
                       sems0, sems1):
    cid = lax.axis_index("c")
    sid = lax.axis_index("s")
    wid = sid * NC + cid

    zeros16 = jnp.zeros((16,), jnp.float32)

    src_v = (src_v0, src_v1)
    dst_v = (dst_v0, dst_v1)
    rows_v = (rows_v0, rows_v1)
    semg = (semg0, semg1)
    semi = (semi0, semi1)
    sems = (sems0, sems1)

    def _fill_z(r, _):
        for c in range(D // 16):
            rows_v0[r, pl.ds(c * 16, 16)] = zeros16
        return 0
    lax.fori_loop(0, CHUNK, _fill_z, 0)

    def _fill_zdeg(r, _):
        deg_loc[pl.ds(r * 16, 16)] = zeros16
        return 0
    lax.fori_loop(0, N_PAD // 16, _fill_zdeg, 0)

    r0 = sid * ROWS_PER_TILE
    ZB = 64
    for j in range(ROWS_PER_TILE // ZB):
        pltpu.sync_copy(rows_v0.at[pl.ds(0, ZB)],
                        acc_sh.at[pl.ds(r0 + j * ZB, ZB)])
    plsc.subcore_barrier()

    base = wid * E_PER_W

    def _start_idx(j, b):
        off = base + j * CHUNK
        pltpu.async_copy(src_hbm.at[pl.ds(off, CHUNK)], src_v[b], semi[b])
        pltpu.async_copy(dst_hbm.at[pl.ds(off, CHUNK)], dst_v[b], semi[b])

    def _wait_idx(b):
        pltpu.make_async_copy(src_hbm.at[pl.ds(0, CHUNK)], src_v[b],
                              semi[b]).wait()
        pltpu.make_async_copy(dst_hbm.at[pl.ds(0, CHUNK)], dst_v[b],
                              semi[b]).wait()

    def _start_gather(b):
        pltpu.async_copy(x_hbm.at[src_v[b]], rows_v[b], semg[b])

    def _wait_gather(b):
        pltpu.make_async_copy(x_hbm.at[src_v[b]], rows_v[b], semg[b]).wait()

    def _finish_chunk(b):
        pltpu.async_copy(rows_v[b], acc_sh.at[dst_v[b]], sems[b], add=True)
        for i in range(CHUNK // 16):
            idx = dst_v[b][pl.ds(i * 16, 16)]
            cnt, last = plsc.scan_count(idx)
            plsc.addupdate_scatter(deg_loc, [idx], cnt.astype(jnp.float32),
                                   mask=last)

    def _wait_scatter(b):
        pltpu.make_async_copy(rows_v[b], acc_sh.at[dst_v[b]],
                              sems[b]).wait()

    _start_idx(0, 0)
    _wait_idx(0)
    _start_gather(0)
    _start_idx(1, 1)

    def _step(j, b, prefetch):
        _wait_gather(b)
        _wait_idx(1 - b)

        @pl.when(j >= 1)
        def _():
            _wait_scatter(1 - b)
        _start_gather(1 - b)
        _finish_chunk(b)

        @pl.when(prefetch)
        def _():
            _start_idx(j + 2, b)

    def _pair(t, _):
        j = 2 * t
        _step(j, 0, jnp.bool_(True))
        _step(j + 1, 1, t < (N_CHUNKS - 1) // 2 - 1)
        return 0

    lax.fori_loop(0, (N_CHUNKS - 1) // 2, _pair, 0)

    _wait_gather(0)
    _finish_chunk(0)
    _wait_scatter(1)
    _wait_scatter(0)
    plsc.subcore_barrier()

    out0 = cid * N_PAD + r0
    for j in range(ROWS_PER_TILE // ZB):
        pltpu.sync_copy(acc_sh.at[pl.ds(r0 + j * ZB, ZB)],
                        rows_v0.at[pl.ds(0, ZB)])
        pltpu.sync_copy(rows_v0.at[pl.ds(0, ZB)],
                        acc_out.at[pl.ds(out0 + j * ZB, ZB)])
    pltpu.sync_copy(deg_loc, deg_out.at[pl.ds(wid * N_PAD, N_PAD)])


_sc_aggregate = pl.kernel(
    _sc_aggregate_body,
    out_type=_SC_OUT_TYPE,
    mesh=_mesh,
    scratch_types=_SC_SCRATCH,
    compiler_params=pltpu.CompilerParams(needs_layout_passes=False),
)


TC_BLK = 1024


def _tc_finish_body(acc_ref, deg_ref, w_ref, b_ref, out_ref):
    a = acc_ref[0] + acc_ref[1]
    d = jnp.sum(deg_ref[...], axis=0)[:, None]
    mean = a / jnp.maximum(d, 1.0)
    out_ref[...] = (
        jnp.dot(mean, w_ref[...], preferred_element_type=jnp.float32)
        + b_ref[...]
    )


def _tc_finish(acc, deg, weight, bias2d):
    grid = (N_PAD // TC_BLK,)
    return pl.pallas_call(
        _tc_finish_body,
        grid=grid,
        in_specs=[
            pl.BlockSpec((NC, TC_BLK, D), lambda i: (0, i, 0)),
            pl.BlockSpec((NW, TC_BLK), lambda i: (0, i)),
            pl.BlockSpec((D, D), lambda i: (0, 0)),
            pl.BlockSpec((1, D), lambda i: (0, 0)),
        ],
        out_specs=pl.BlockSpec((TC_BLK, D), lambda i: (i, 0)),
        out_shape=jax.ShapeDtypeStruct((N_PAD, D), jnp.float32),
    )(acc, deg, weight, bias2d)


@jax.jit
def kernel(x, edge_index, weight, bias):
    npad = E_PAD - N_EDGES
    if npad:
        dst = jnp.concatenate(
            [edge_index[0], jnp.full((npad,), N_NODES, jnp.int32)])
        src = jnp.concatenate([edge_index[1], jnp.zeros((npad,), jnp.int32)])
    else:
        dst = edge_index[0]
        src = edge_index[1]
    acc, deg = _sc_aggregate(src, dst, x)
    acc = acc.reshape(NC, N_PAD, D)
    deg = deg.reshape(NW, N_PAD)
    out = _tc_finish(acc, deg, weight, bias.reshape(1, D))
    return out[:N_NODES]


# speedup vs baseline: 1.5700x; 1.0210x over previous
import functools

import jax
import jax.numpy as jnp
from jax import lax
from jax.experimental import pallas as pl
from jax.experimental.pallas import tpu as pltpu
from jax.experimental.pallas import tpu_sc as plsc

N_NODES = 10000
N_EDGES = 320000
D = 128

NC = 2
NS = 16
NW = NC * NS
CHUNK = 80
N_CHUNKS = 125
E_PER_W = N_CHUNKS * CHUNK
E_PAD = NW * E_PER_W
N_PAD = 10240
ROWS_PER_TILE = N_PAD // NS

_mesh = plsc.VectorSubcoreMesh(core_axis_name="c", subcore_axis_name="s")

_SC_OUT_TYPE = [
    jax.ShapeDtypeStruct((NC * N_PAD, D), jnp.float32),
    jax.ShapeDtypeStruct((NW * N_PAD,), jnp.float32),
]
_SC_SCRATCH = [
    pltpu.VMEM((CHUNK,), jnp.int32),
    pltpu.VMEM((CHUNK,), jnp.int32),
    pltpu.VMEM((CHUNK,), jnp.int32),
    pltpu.VMEM((CHUNK,), jnp.int32),
    pltpu.VMEM((CHUNK, D), jnp.float32),
    pltpu.VMEM((CHUNK, D), jnp.float32),
    pltpu.VMEM((N_PAD,), jnp.float32),
    pltpu.VMEM_SHARED((N_PAD, D), jnp.float32),
    pltpu.SemaphoreType.DMA,
    pltpu.SemaphoreType.DMA,
    pltpu.SemaphoreType.DMA,
    pltpu.SemaphoreType.DMA,
    pltpu.SemaphoreType.DMA,
    pltpu.SemaphoreType.DMA,
]


def _sc_aggregate_body(src_hbm, dst_hbm, x_hbm, acc_out, deg_out,
                       src_v0, dst_v0, src_v1, dst_v1, rows_v0, rows_v1,
                       deg_loc, acc_sh, semg0, semg1, semi0, semi1,
                       sems0, sems1):
    cid = lax.axis_index("c")
    sid = lax.axis_index("s")
    wid = sid * NC + cid

    zeros16 = jnp.zeros((16,), jnp.float32)

    src_v = (src_v0, src_v1)
    dst_v = (dst_v0, dst_v1)
    rows_v = (rows_v0, rows_v1)
    semg = (semg0, semg1)
    semi = (semi0, semi1)
    sems = (sems0, sems1)

    def _fill_z(r, _):
        for c in range(D // 16):
            rows_v0[r, pl.ds(c * 16, 16)] = zeros16
        return 0
    lax.fori_loop(0, CHUNK, _fill_z, 0)

    def _fill_zdeg(r, _):
        deg_loc[pl.ds(r * 16, 16)] = zeros16
        return 0
    lax.fori_loop(0, N_PAD // 16, _fill_zdeg, 0)

    r0 = sid * ROWS_PER_TILE
    ZB = 64
    for j in range(ROWS_PER_TILE // ZB):
        pltpu.sync_copy(rows_v0.at[pl.ds(0, ZB)],
                        acc_sh.at[pl.ds(r0 + j * ZB, ZB)])
    plsc.subcore_barrier()

    base = wid * E_PER_W

    def _start_idx(j, b):
        off = base + j * CHUNK
        pltpu.async_copy(src_hbm.at[pl.ds(off, CHUNK)], src_v[b], semi[b])
        pltpu.async_copy(dst_hbm.at[pl.ds(off, CHUNK)], dst_v[b], semi[b])

    def _wait_idx(b):
        pltpu.make_async_copy(src_hbm.at[pl.ds(0, CHUNK)], src_v[b],
                              semi[b]).wait()
        pltpu.make_async_copy(dst_hbm.at[pl.ds(0, CHUNK)], dst_v[b],
                              semi[b]).wait()

    def _start_gather(b):
        pltpu.async_copy(x_hbm.at[src_v[b]], rows_v[b], semg[b])

    def _wait_gather(b):
        pltpu.make_async_copy(x_hbm.at[src_v[b]], rows_v[b], semg[b]).wait()

    def _finish_chunk(b):
        pltpu.async_copy(rows_v[b], acc_sh.at[dst_v[b]], sems[b], add=True)
        for i in range(CHUNK // 16):
            idx = dst_v[b][pl.ds(i * 16, 16)]
            cnt, last = plsc.scan_count(idx)
            plsc.addupdate_scatter(deg_loc, [idx], cnt.astype(jnp.float32),
                                   mask=last)

    def _wait_scatter(b):
        pltpu.make_async_copy(rows_v[b], acc_sh.at[dst_v[b]],
                              sems[b]).wait()

    _start_idx(0, 0)
    _wait_idx(0)
    _start_gather(0)
    _start_idx(1, 1)

    def _step(j, b, prefetch):
        _wait_gather(b)
        _wait_idx(1 - b)

        @pl.when(j >= 1)
        def _():
            _wait_scatter(1 - b)
        _start_gather(1 - b)
        _finish_chunk(b)

        @pl.when(prefetch)
        def _():
            _start_idx(j + 2, b)

    def _pair(t, _):
        j = 2 * t
        _step(j, 0, jnp.bool_(True))
        _step(j + 1, 1, t < (N_CHUNKS - 1) // 2 - 1)
        return 0

    lax.fori_loop(0, (N_CHUNKS - 1) // 2, _pair, 0)

    _wait_gather(0)
    _finish_chunk(0)
    _wait_scatter(1)
    _wait_scatter(0)
    plsc.subcore_barrier()

    out0 = cid * N_PAD + r0
    for j in range(ROWS_PER_TILE // ZB):
        pltpu.sync_copy(acc_sh.at[pl.ds(r0 + j * ZB, ZB)],
                        rows_v0.at[pl.ds(0, ZB)])
        pltpu.sync_copy(rows_v0.at[pl.ds(0, ZB)],
                        acc_out.at[pl.ds(out0 + j * ZB, ZB)])
    pltpu.sync_copy(deg_loc, deg_out.at[pl.ds(wid * N_PAD, N_PAD)])


_sc_aggregate = pl.kernel(
    _sc_aggregate_body,
    out_type=_SC_OUT_TYPE,
    mesh=_mesh,
    scratch_types=_SC_SCRATCH,
    compiler_params=pltpu.CompilerParams(needs_layout_passes=False),
)


TC_BLK = 1024


def _tc_finish_body(acc_ref, deg_ref, w_ref, b_ref, out_ref):
    a = acc_ref[0] + acc_ref[1]
    d = jnp.sum(deg_ref[...], axis=0)[:, None]
    mean = a / jnp.maximum(d, 1.0)
    out_ref[...] = (
        jnp.dot(mean, w_ref[...], preferred_element_type=jnp.float32)
        + b_ref[...]
    )


def _tc_finish(acc, deg, weight, bias2d):
    grid = (N_PAD // TC_BLK,)
    return pl.pallas_call(
        _tc_finish_body,
        grid=grid,
        in_specs=[
            pl.BlockSpec((NC, TC_BLK, D), lambda i: (0, i, 0)),
            pl.BlockSpec((NW, TC_BLK), lambda i: (0, i)),
            pl.BlockSpec((D, D), lambda i: (0, 0)),
            pl.BlockSpec((1, D), lambda i: (0, 0)),
        ],
        out_specs=pl.BlockSpec((TC_BLK, D), lambda i: (i, 0)),
        out_shape=jax.ShapeDtypeStruct((N_NODES, D), jnp.float32),
    )(acc, deg, weight, bias2d)


@jax.jit
def kernel(x, edge_index, weight, bias):
    npad = E_PAD - N_EDGES
    if npad:
        dst = jnp.concatenate(
            [edge_index[0], jnp.full((npad,), N_NODES, jnp.int32)])
        src = jnp.concatenate([edge_index[1], jnp.zeros((npad,), jnp.int32)])
    else:
        dst = edge_index[0]
        src = edge_index[1]
    acc, deg = _sc_aggregate(src, dst, x)
    acc = acc.reshape(NC, N_PAD, D)
    deg = deg.reshape(NW, N_PAD)
    return _tc_finish(acc, deg, weight, bias.reshape(1, D))
